# Initial kernel scaffold; baseline (speedup 1.0000x reference)
import functools

import jax
import jax.numpy as jnp
from jax import lax
from jax.experimental import pallas as pl
from jax.experimental.pallas import tpu as pltpu
from jax.experimental.pallas import tpu_sc as plsc

_N = 10000
_E = 320000
_G = 64
_NC = 2
_NS = 16
_NW = _NC * _NS
_EPW = _E // _NW
_CH = 80
_NCHUNK = _EPW // _CH
_RPS = _N // _NS
_ZR = 125
_BLK = 400
_NBLK = _N // _BLK


def _vector_mesh():
    return plsc.VectorSubcoreMesh(core_axis_name="c", subcore_axis_name="s")


def _sc_degree(col, ew):
    @functools.partial(
        pl.kernel,
        out_type=jax.ShapeDtypeStruct((_NC, _N, 16), jnp.float32),
        mesh=_vector_mesh(),
        scratch_types=[
            pltpu.VMEM((_CH,), jnp.int32),
            pltpu.VMEM((_CH,), jnp.float32),
            pltpu.VMEM((_CH, 16), jnp.float32),
            pltpu.VMEM((_ZR, 16), jnp.float32),
            pltpu.VMEM_SHARED((_N, 16), jnp.float32),
        ],
    )
    def deg_kernel(col_hbm, ew_hbm, out_hbm, colv, ewv, srcv, zbuf, acc):
        cid = lax.axis_index("c")
        sid = lax.axis_index("s")
        wid = sid * _NC + cid
        zero16 = jnp.zeros((16,), jnp.float32)

        @pl.loop(0, _ZR)
        def _(r):
            zbuf[r, pl.ds(0, 16)] = zero16

        @pl.loop(0, _CH)
        def _(r):
            srcv[r, pl.ds(0, 16)] = zero16

        for j in range(_RPS // _ZR):
            pltpu.sync_copy(zbuf, acc.at[pl.ds(sid * _RPS + j * _ZR, _ZR)])
        plsc.subcore_barrier()

        lane0 = jnp.zeros((16,), jnp.int32)
        riota = lax.broadcasted_iota(jnp.int32, (16,), 0)

        @pl.loop(0, _NCHUNK)
        def _(ch):
            base = wid * _EPW + ch * _CH
            pltpu.sync_copy(col_hbm.at[pl.ds(base, _CH)], colv)
            pltpu.sync_copy(ew_hbm.at[pl.ds(base, _CH)], ewv)
            for kk in range(_CH // 16):
                v = ewv[pl.ds(kk * 16, 16)]
                plsc.store_scatter(srcv, [riota + kk * 16, lane0], v)
            pltpu.sync_copy(srcv, acc.at[colv], add=True)

        plsc.subcore_barrier()
        for j in range(_RPS // _ZR):
            off = sid * _RPS + j * _ZR
            pltpu.sync_copy(acc.at[pl.ds(off, _ZR)], out_hbm.at[cid, pl.ds(off, _ZR)])

    return deg_kernel(col, ew)


def _sc_conv(table, row, col, ew, d):
    @functools.partial(
        pl.kernel,
        out_type=jax.ShapeDtypeStruct((_NC, _N, d), jnp.float32),
        mesh=_vector_mesh(),
        scratch_types=[
            pltpu.VMEM((_CH,), jnp.int32),
            pltpu.VMEM((_CH,), jnp.int32),
            pltpu.VMEM((_CH,), jnp.float32),
            pltpu.VMEM((_CH, d), jnp.float32),
            pltpu.VMEM((_ZR, d), jnp.float32),
            pltpu.VMEM_SHARED((_N, d), jnp.float32),
            pltpu.SemaphoreType.DMA,
        ],
    )
    def conv_kernel(table_hbm, row_hbm, col_hbm, ew_hbm, out_hbm,
                    rowv, colv, ewv, msg, zbuf, acc, sem):
        cid = lax.axis_index("c")
        sid = lax.axis_index("s")
        wid = sid * _NC + cid
        zero16 = jnp.zeros((16,), jnp.float32)

        @pl.loop(0, _ZR)
        def _(r):
            for kk in range(d // 16):
                zbuf[r, pl.ds(kk * 16, 16)] = zero16

        for j in range(_RPS // _ZR):
            pltpu.sync_copy(zbuf, acc.at[pl.ds(sid * _RPS + j * _ZR, _ZR)])
        plsc.subcore_barrier()

        @pl.loop(0, _NCHUNK)
        def _(ch):
            base = wid * _EPW + ch * _CH
            pltpu.sync_copy(row_hbm.at[pl.ds(base, _CH)], rowv)
            pltpu.sync_copy(col_hbm.at[pl.ds(base, _CH)], colv)
            pltpu.sync_copy(ew_hbm.at[pl.ds(base, _CH)], ewv)
            pltpu.async_copy(table_hbm.at[rowv], msg, sem).wait()

            @pl.loop(0, _CH)
            def _(e):
                w = plsc.load_gather(ewv, [jnp.zeros((16,), jnp.int32) + e])
                for kk in range(d // 16):
                    msg[e, pl.ds(kk * 16, 16)] = msg[e, pl.ds(kk * 16, 16)] * w

            pltpu.sync_copy(msg, acc.at[colv], add=True)

        plsc.subcore_barrier()
        for j in range(_RPS // _ZR):
            off = sid * _RPS + j * _ZR
            pltpu.sync_copy(acc.at[pl.ds(off, _ZR)], out_hbm.at[cid, pl.ds(off, _ZR)])

    return conv_kernel(table, row, col, ew)


def _dinv_block(d0_ref, d1_ref):
    deg = d0_ref[:, 0:1] + d1_ref[:, 0:1] + 1.0
    return lax.rsqrt(deg)


def _tc_stage1(x, d0, d1, w1):
    def body(x_ref, d0_ref, d1_ref, w_ref, o_ref):
        dinv = _dinv_block(d0_ref, d1_ref)
        xw = jnp.dot(x_ref[...], w_ref[...], preferred_element_type=jnp.float32)
        o_ref[...] = xw * dinv

    return pl.pallas_call(
        body,
        grid=(_NBLK,),
        in_specs=[
            pl.BlockSpec((_BLK, 128), lambda i: (i, 0)),
            pl.BlockSpec((_BLK, 16), lambda i: (i, 0)),
            pl.BlockSpec((_BLK, 16), lambda i: (i, 0)),
            pl.BlockSpec((128, 128), lambda i: (0, 0)),
        ],
        out_specs=pl.BlockSpec((_BLK, 128), lambda i: (i, 0)),
        out_shape=jax.ShapeDtypeStruct((_N, 128), jnp.float32),
    )(x, d0, d1, w1)


def _tc_stage2(a0, a1, t1, d0, d1, b1, w2):
    def body(a0_ref, a1_ref, t1_ref, d0_ref, d1_ref, b_ref, w_ref, o_ref):
        dinv = _dinv_block(d0_ref, d1_ref)
        h = dinv * (a0_ref[...] + a1_ref[...] + t1_ref[...]) + b_ref[...]
        h = jnp.maximum(h, 0.0)
        hw = jnp.dot(h, w_ref[...], preferred_element_type=jnp.float32)
        o_ref[...] = hw * dinv

    return pl.pallas_call(
        body,
        grid=(_NBLK,),
        in_specs=[
            pl.BlockSpec((_BLK, 128), lambda i: (i, 0)),
            pl.BlockSpec((_BLK, 128), lambda i: (i, 0)),
            pl.BlockSpec((_BLK, 128), lambda i: (i, 0)),
            pl.BlockSpec((_BLK, 16), lambda i: (i, 0)),
            pl.BlockSpec((_BLK, 16), lambda i: (i, 0)),
            pl.BlockSpec((1, 128), lambda i: (0, 0)),
            pl.BlockSpec((128, 64), lambda i: (0, 0)),
        ],
        out_specs=pl.BlockSpec((_BLK, 64), lambda i: (i, 0)),
        out_shape=jax.ShapeDtypeStruct((_N, 64), jnp.float32),
    )(a0, a1, t1, d0, d1, b1, w2)


def _tc_final(a0, a1, t2, d0, d1, b2, batch3, wl1, bl1, wl2, bl2):
    def body(a0_ref, a1_ref, t2_ref, d0_ref, d1_ref, b_ref, batch_ref,
             wl1_ref, bl1_ref, wl2_ref, bl2_ref,
             yp_ref, yh_ref, out_ref, pooled_ref):
        i = pl.program_id(0)

        @pl.when(i == 0)
        def _():
            pooled_ref[...] = jnp.zeros_like(pooled_ref)

        dinv = _dinv_block(d0_ref, d1_ref)
        h2 = dinv * (a0_ref[...] + a1_ref[...] + t2_ref[...]) + b_ref[...]
        h2 = jnp.maximum(h2, 0.0)
        b = batch_ref[...][0]
        gids = lax.broadcasted_iota(jnp.int32, (_G, _BLK), 0)
        onehot = (jnp.broadcast_to(b, (_G, _BLK)) == gids).astype(jnp.float32)
        pooled_ref[...] += jnp.dot(onehot, h2, preferred_element_type=jnp.float32)

        @pl.when(i == _NBLK - 1)
        def _():
            p = pooled_ref[...]
            h3 = jnp.dot(p, wl1_ref[...], preferred_element_type=jnp.float32)
            h3 = jnp.maximum(h3 + bl1_ref[...], 0.0)
            o = jnp.dot(h3, wl2_ref[...], preferred_element_type=jnp.float32)
            o = o + bl2_ref[...]
            m = jnp.max(o, axis=1, keepdims=True)
            lse = m + jnp.log(jnp.sum(jnp.exp(o - m), axis=1, keepdims=True))
            yp_ref[...] = o - lse
            yh_ref[...] = (o[:, 1:2] > o[:, 0:1]).astype(jnp.float32)
            out_ref[...] = o

    return pl.pallas_call(
        body,
        grid=(_NBLK,),
        in_specs=[
            pl.BlockSpec((_BLK, 64), lambda i: (i, 0)),
            pl.BlockSpec((_BLK, 64), lambda i: (i, 0)),
            pl.BlockSpec((_BLK, 64), lambda i: (i, 0)),
            pl.BlockSpec((_BLK, 16), lambda i: (i, 0)),
            pl.BlockSpec((_BLK, 16), lambda i: (i, 0)),
            pl.BlockSpec((1, 64), lambda i: (0, 0)),
            pl.BlockSpec((1, 1, _BLK), lambda i: (i, 0, 0)),
            pl.BlockSpec((64, 64), lambda i: (0, 0)),
            pl.BlockSpec((1, 64), lambda i: (0, 0)),
            pl.BlockSpec((64, 2), lambda i: (0, 0)),
            pl.BlockSpec((1, 2), lambda i: (0, 0)),
        ],
        out_specs=[
            pl.BlockSpec((_G, 2), lambda i: (0, 0)),
            pl.BlockSpec((_G, 1), lambda i: (0, 0)),
            pl.BlockSpec((_G, 2), lambda i: (0, 0)),
        ],
        out_shape=[
            jax.ShapeDtypeStruct((_G, 2), jnp.float32),
            jax.ShapeDtypeStruct((_G, 1), jnp.float32),
            jax.ShapeDtypeStruct((_G, 2), jnp.float32),
        ],
        scratch_shapes=[pltpu.VMEM((_G, 64), jnp.float32)],
    )(a0, a1, t2, d0, d1, b2, batch3, wl1, bl1, wl2, bl2)


def kernel(x, edge_index, edge_weight, batch, W1, b1, W2, b2, Wl1, bl1, Wl2, bl2):
    row = edge_index[0]
    col = edge_index[1]

    deg16 = _sc_degree(col, edge_weight)
    d0 = deg16[0]
    d1 = deg16[1]

    table1 = _tc_stage1(x, d0, d1, W1)
    agg1 = _sc_conv(table1, row, col, edge_weight, 128)
    table2 = _tc_stage2(agg1[0], agg1[1], table1, d0, d1,
                        b1.reshape(1, 128), W2)
    agg2 = _sc_conv(table2, row, col, edge_weight, 64)

    batch3 = batch.reshape(_NBLK, 1, _BLK)
    y_prob, y_hat, out = _tc_final(agg2[0], agg2[1], table2, d0, d1,
                                   b2.reshape(1, 64), batch3,
                                   Wl1, bl1.reshape(1, 64),
                                   Wl2, bl2.reshape(1, 2))
    return (y_prob, y_hat.reshape(_G), out)


# speedup vs baseline: 7.9809x; 7.9809x over previous
import dataclasses
import functools

import jax
import jax.numpy as jnp
from jax import lax
from jax.experimental import pallas as pl
from jax.experimental.pallas import tpu as pltpu
from jax.experimental.pallas import tpu_sc as plsc

_N = 10000
_E = 320000
_G = 64
_NC = 2
_NS = 16
_NW = _NC * _NS
_EPW = _E // _NW
_CH = 80
_NCHUNK = _EPW // _CH
_NPAD = 10240
_RPS = _NPAD // _NS
_ZR = 128
_BLK = 400
_NBLK = _N // _BLK


def _vector_mesh():
    return plsc.VectorSubcoreMesh(core_axis_name="c", subcore_axis_name="s")


def _sc_params():
    cp = pltpu.CompilerParams()
    if "needs_layout_passes" in pltpu.CompilerParams.__dataclass_fields__:
        cp = dataclasses.replace(cp, needs_layout_passes=False)
    return cp


def _sc_degree(col, ew):
    @functools.partial(
        pl.kernel,
        out_type=jax.ShapeDtypeStruct((_NC, _NPAD, 128), jnp.float32),
        mesh=_vector_mesh(),
        compiler_params=_sc_params(),
        scratch_types=[
            pltpu.VMEM((_CH,), jnp.int32),
            pltpu.VMEM((_CH,), jnp.float32),
            pltpu.VMEM((_CH, 128), jnp.float32),
            pltpu.VMEM((_ZR, 128), jnp.float32),
            pltpu.VMEM_SHARED((_NPAD, 128), jnp.float32),
        ],
    )
    def deg_kernel(col_hbm, ew_hbm, out_hbm, colv, ewv, srcv, zbuf, acc):
        cid = lax.axis_index("c")
        sid = lax.axis_index("s")
        wid = sid * _NC + cid
        zero16 = jnp.zeros((16,), jnp.float32)

        @pl.loop(0, _ZR)
        def _(r):
            for kk in range(8):
                zbuf[r, pl.ds(kk * 16, 16)] = zero16

        @pl.loop(0, _CH)
        def _(r):
            for kk in range(8):
                srcv[r, pl.ds(kk * 16, 16)] = zero16

        for j in range(_RPS // _ZR):
            pltpu.sync_copy(zbuf, acc.at[pl.ds(sid * _RPS + j * _ZR, _ZR)])
        plsc.subcore_barrier()

        @pl.loop(0, _NCHUNK)
        def _(ch):
            base = wid * _EPW + ch * _CH
            pltpu.sync_copy(col_hbm.at[pl.ds(base, _CH)], colv)
            pltpu.sync_copy(ew_hbm.at[pl.ds(base, _CH)], ewv)

            @pl.loop(0, _CH)
            def _(e):
                w = plsc.load_gather(ewv, [jnp.zeros((16,), jnp.int32) + e])
                srcv[e, pl.ds(0, 16)] = w

            pltpu.sync_copy(srcv, acc.at[colv], add=True)

        plsc.subcore_barrier()
        for j in range(_RPS // _ZR):
            off = sid * _RPS + j * _ZR
            pltpu.sync_copy(acc.at[pl.ds(off, _ZR)], out_hbm.at[cid, pl.ds(off, _ZR)])

    return deg_kernel(col, ew)


def _sc_conv(table, row, col, ew, d):
    @functools.partial(
        pl.kernel,
        out_type=jax.ShapeDtypeStruct((_NC, _NPAD, d), jnp.float32),
        mesh=_vector_mesh(),
        compiler_params=_sc_params(),
        scratch_types=[
            pltpu.VMEM((_CH,), jnp.int32),
            pltpu.VMEM((_CH,), jnp.int32),
            pltpu.VMEM((_CH,), jnp.float32),
            pltpu.VMEM((_CH, d), jnp.float32),
            pltpu.VMEM((_ZR, d), jnp.float32),
            pltpu.VMEM_SHARED((_NPAD, d), jnp.float32),
            pltpu.SemaphoreType.DMA,
        ],
    )
    def conv_kernel(table_hbm, row_hbm, col_hbm, ew_hbm, out_hbm,
                    rowv, colv, ewv, msg, zbuf, acc, sem):
        cid = lax.axis_index("c")
        sid = lax.axis_index("s")
        wid = sid * _NC + cid
        zero16 = jnp.zeros((16,), jnp.float32)

        @pl.loop(0, _ZR)
        def _(r):
            for kk in range(d // 16):
                zbuf[r, pl.ds(kk * 16, 16)] = zero16

        for j in range(_RPS // _ZR):
            pltpu.sync_copy(zbuf, acc.at[pl.ds(sid * _RPS + j * _ZR, _ZR)])
        plsc.subcore_barrier()

        @pl.loop(0, _NCHUNK)
        def _(ch):
            base = wid * _EPW + ch * _CH
            pltpu.sync_copy(row_hbm.at[pl.ds(base, _CH)], rowv)
            pltpu.sync_copy(col_hbm.at[pl.ds(base, _CH)], colv)
            pltpu.sync_copy(ew_hbm.at[pl.ds(base, _CH)], ewv)
            pltpu.async_copy(table_hbm.at[rowv], msg, sem).wait()

            @pl.loop(0, _CH)
            def _(e):
                w = plsc.load_gather(ewv, [jnp.zeros((16,), jnp.int32) + e])
                for kk in range(d // 16):
                    msg[e, pl.ds(kk * 16, 16)] = msg[e, pl.ds(kk * 16, 16)] * w

            pltpu.sync_copy(msg, acc.at[colv], add=True)

        plsc.subcore_barrier()
        for j in range(_RPS // _ZR):
            off = sid * _RPS + j * _ZR
            pltpu.sync_copy(acc.at[pl.ds(off, _ZR)], out_hbm.at[cid, pl.ds(off, _ZR)])

    return conv_kernel(table, row, col, ew)


def _dinv_block(d0_ref, d1_ref):
    deg = d0_ref[:, 0:1] + d1_ref[:, 0:1] + 1.0
    return lax.rsqrt(deg)


def _tc_stage1(x, d0, d1, w1):
    def body(x_ref, d0_ref, d1_ref, w_ref, o_ref):
        dinv = _dinv_block(d0_ref, d1_ref)
        xw = jnp.dot(x_ref[...], w_ref[...], preferred_element_type=jnp.float32)
        o_ref[...] = xw * dinv

    return pl.pallas_call(
        body,
        grid=(_NBLK,),
        in_specs=[
            pl.BlockSpec((_BLK, 128), lambda i: (i, 0)),
            pl.BlockSpec((_BLK, 128), lambda i: (i, 0)),
            pl.BlockSpec((_BLK, 128), lambda i: (i, 0)),
            pl.BlockSpec((128, 128), lambda i: (0, 0)),
        ],
        out_specs=pl.BlockSpec((_BLK, 128), lambda i: (i, 0)),
        out_shape=jax.ShapeDtypeStruct((_N, 128), jnp.float32),
    )(x, d0, d1, w1)


def _tc_stage2(a0, a1, t1, d0, d1, b1, w2):
    def body(a0_ref, a1_ref, t1_ref, d0_ref, d1_ref, b_ref, w_ref, o_ref):
        dinv = _dinv_block(d0_ref, d1_ref)
        h = dinv * (a0_ref[...] + a1_ref[...] + t1_ref[...]) + b_ref[...]
        h = jnp.maximum(h, 0.0)
        hw = jnp.dot(h, w_ref[...], preferred_element_type=jnp.float32)
        o_ref[...] = hw * dinv

    return pl.pallas_call(
        body,
        grid=(_NBLK,),
        in_specs=[
            pl.BlockSpec((_BLK, 128), lambda i: (i, 0)),
            pl.BlockSpec((_BLK, 128), lambda i: (i, 0)),
            pl.BlockSpec((_BLK, 128), lambda i: (i, 0)),
            pl.BlockSpec((_BLK, 128), lambda i: (i, 0)),
            pl.BlockSpec((_BLK, 128), lambda i: (i, 0)),
            pl.BlockSpec((1, 128), lambda i: (0, 0)),
            pl.BlockSpec((128, 128), lambda i: (0, 0)),
        ],
        out_specs=pl.BlockSpec((_BLK, 128), lambda i: (i, 0)),
        out_shape=jax.ShapeDtypeStruct((_N, 128), jnp.float32),
    )(a0, a1, t1, d0, d1, b1, w2)


def _tc_final(a0, a1, t2, d0, d1, b2, batch3, wl1, bl1, wl2, bl2):
    def body(a0_ref, a1_ref, t2_ref, d0_ref, d1_ref, b_ref, batch_ref,
             wl1_ref, bl1_ref, wl2_ref, bl2_ref,
             yp_ref, yh_ref, out_ref, pooled_ref):
        i = pl.program_id(0)

        @pl.when(i == 0)
        def _():
            pooled_ref[...] = jnp.zeros_like(pooled_ref)

        dinv = _dinv_block(d0_ref, d1_ref)
        h2 = dinv * (a0_ref[...] + a1_ref[...] + t2_ref[...]) + b_ref[...]
        h2 = jnp.maximum(h2, 0.0)
        b = batch_ref[...][0]
        gids = lax.broadcasted_iota(jnp.int32, (_G, _BLK), 0)
        onehot = (jnp.broadcast_to(b, (_G, _BLK)) == gids).astype(jnp.float32)
        pooled_ref[...] += jnp.dot(onehot, h2, preferred_element_type=jnp.float32)

        @pl.when(i == _NBLK - 1)
        def _():
            p = pooled_ref[...]
            h3 = jnp.dot(p, wl1_ref[...], preferred_element_type=jnp.float32)
            h3 = jnp.maximum(h3 + bl1_ref[...], 0.0)
            o = jnp.dot(h3, wl2_ref[...], preferred_element_type=jnp.float32)
            o = o + bl2_ref[...]
            m = jnp.max(o, axis=1, keepdims=True)
            lse = m + jnp.log(jnp.sum(jnp.exp(o - m), axis=1, keepdims=True))
            yp_ref[...] = o - lse
            yh_ref[...] = (o[:, 1:2] > o[:, 0:1]).astype(jnp.float32)
            out_ref[...] = o

    return pl.pallas_call(
        body,
        grid=(_NBLK,),
        in_specs=[
            pl.BlockSpec((_BLK, 128), lambda i: (i, 0)),
            pl.BlockSpec((_BLK, 128), lambda i: (i, 0)),
            pl.BlockSpec((_BLK, 128), lambda i: (i, 0)),
            pl.BlockSpec((_BLK, 128), lambda i: (i, 0)),
            pl.BlockSpec((_BLK, 128), lambda i: (i, 0)),
            pl.BlockSpec((1, 128), lambda i: (0, 0)),
            pl.BlockSpec((1, 1, _BLK), lambda i: (i, 0, 0)),
            pl.BlockSpec((128, 64), lambda i: (0, 0)),
            pl.BlockSpec((1, 64), lambda i: (0, 0)),
            pl.BlockSpec((64, 2), lambda i: (0, 0)),
            pl.BlockSpec((1, 2), lambda i: (0, 0)),
        ],
        out_specs=[
            pl.BlockSpec((_G, 2), lambda i: (0, 0)),
            pl.BlockSpec((_G, 1), lambda i: (0, 0)),
            pl.BlockSpec((_G, 2), lambda i: (0, 0)),
        ],
        out_shape=[
            jax.ShapeDtypeStruct((_G, 2), jnp.float32),
            jax.ShapeDtypeStruct((_G, 1), jnp.float32),
            jax.ShapeDtypeStruct((_G, 2), jnp.float32),
        ],
        scratch_shapes=[pltpu.VMEM((_G, 128), jnp.float32)],
    )(a0, a1, t2, d0, d1, b2, batch3, wl1, bl1, wl2, bl2)


def kernel(x, edge_index, edge_weight, batch, W1, b1, W2, b2, Wl1, bl1, Wl2, bl2):
    row = edge_index[0]
    col = edge_index[1]

    deg16 = _sc_degree(col, edge_weight)
    d0 = deg16[0]
    d1 = deg16[1]

    w2p = jnp.pad(W2, ((0, 0), (0, 64)))
    b2p = jnp.pad(b2, (0, 64)).reshape(1, 128)
    wl1p = jnp.pad(Wl1, ((0, 64), (0, 0)))

    table1 = _tc_stage1(x, d0, d1, W1)
    agg1 = _sc_conv(table1, row, col, edge_weight, 128)
    table2 = _tc_stage2(agg1[0], agg1[1], table1, d0, d1,
                        b1.reshape(1, 128), w2p)
    agg2 = _sc_conv(table2, row, col, edge_weight, 128)

    batch3 = batch.reshape(_NBLK, 1, _BLK)
    y_prob, y_hat, out = _tc_final(agg2[0], agg2[1], table2, d0, d1,
                                   b2p, batch3,
                                   wl1p, bl1.reshape(1, 64),
                                   Wl2, bl2.reshape(1, 2))
    return (y_prob, y_hat.reshape(_G), out)
